# Initial kernel scaffold; baseline (speedup 1.0000x reference)
import jax
import jax.numpy as jnp
from jax.experimental import pallas as pl


def kernel(s, i, j, W1, b1, W2, b2, W3, b3):
    raise NotImplementedError("write your pallas kernel here")


# speedup vs baseline: 1.6048x; 1.6048x over previous
import functools

import jax
import jax.numpy as jnp
from jax.experimental import pallas as pl
from jax.experimental.pallas import tpu as pltpu

_BE = 12800


def _mlp_body(d_ref, w1_ref, b1_ref, w2_ref, b2_ref, w3_ref, b3_ref,
              ox_ref, oy_ref):
    D = d_ref[...]
    dx = D[0:1, :]
    dy = D[1:2, :]
    vx = D[2:3, :]
    vy = D[3:4, :]
    d2 = dx * dx + dy * dy
    dist = jnp.maximum(jnp.sqrt(d2), 1e-8)
    inv = 1.0 / dist
    nx = dx * inv
    ny = dy * inv
    app = (dx * vx + dy * vy) * inv
    X = jnp.concatenate([dist, app], axis=0)
    z1 = jax.lax.dot_general(w1_ref[...], X, (((0,), (0,)), ((), ())),
                             preferred_element_type=jnp.float32)
    z1 = z1 + b1_ref[...]
    h1 = z1 * (1.0 / (1.0 + jnp.exp(-z1)))
    z2 = jax.lax.dot_general(w2_ref[...], h1, (((0,), (0,)), ((), ())),
                             preferred_element_type=jnp.float32)
    z2 = z2 + b2_ref[...]
    h2 = z2 * (1.0 / (1.0 + jnp.exp(-z2)))
    f = jax.lax.dot_general(w3_ref[...], h2, (((0,), (0,)), ((), ())),
                            preferred_element_type=jnp.float32)
    f = f + b3_ref[...]
    ox_ref[...] = f * nx
    oy_ref[...] = f * ny


def _edge_mlp(Dm, W1, b1, W2, b2, W3, b3):
    E = Dm.shape[1]
    be = _BE if E % _BE == 0 else E
    grid = E // be

    def _w(shape):
        return pl.BlockSpec(shape, lambda e: (0, 0))

    return pl.pallas_call(
        _mlp_body,
        grid=(grid,),
        in_specs=[
            pl.BlockSpec((4, be), lambda e: (0, e)),
            _w((2, 16)), _w((16, 1)), _w((16, 16)), _w((16, 1)),
            _w((16, 1)), _w((1, 1)),
        ],
        out_specs=[
            pl.BlockSpec((1, be), lambda e: (0, e)),
            pl.BlockSpec((1, be), lambda e: (0, e)),
        ],
        out_shape=[
            jax.ShapeDtypeStruct((1, E), jnp.float32),
            jax.ShapeDtypeStruct((1, E), jnp.float32),
        ],
    )(Dm, W1, b1.reshape(16, 1), W2, b2.reshape(16, 1), W3,
      b3.reshape(1, 1))


def kernel(s, i, j, W1, b1, W2, b2, W3, b3):
    p = s[0]
    diff = jnp.take(p, j, axis=0) - jnp.take(p, i, axis=0)
    Dm = diff.T
    impx, impy = _edge_mlp(Dm, W1, b1, W2, b2, W3, b3)
    imp = jnp.stack([impx[0], impy[0]], axis=-1)
    vel = s[:, :, 2:]
    v_new = vel.at[:, i].add(imp[None])
    v_new = v_new.at[:, j].add(-imp[None])
    return jnp.concatenate([s[:, :, :2], v_new], axis=-1)


# speedup vs baseline: 21.8252x; 13.5998x over previous
import functools

import jax
import jax.numpy as jnp
from jax import lax
from jax.experimental import pallas as pl
from jax.experimental.pallas import tpu as pltpu
from jax.experimental.pallas import tpu_sc as plsc

_BE = 12800
_C1 = 6400
_NW = 32


def _sc_gather(T1, i4, j4):
    E = i4.shape[0]
    C = _C1
    nchunks = E // C
    full, extra = divmod(nchunks, _NW)
    mesh = plsc.VectorSubcoreMesh(core_axis_name="c", subcore_axis_name="s")

    @functools.partial(
        pl.kernel,
        mesh=mesh,
        compiler_params=pltpu.CompilerParams(use_tc_tiling_on_sc=False),
        out_type=jax.ShapeDtypeStruct((4 * E,), jnp.float32),
        scratch_types=[
            pltpu.VMEM((C,), jnp.int32),
            pltpu.VMEM((C,), jnp.int32),
            pltpu.VMEM((C,), jnp.int32),
            pltpu.VMEM((C,), jnp.int32),
            pltpu.VMEM((C,), jnp.float32),
            pltpu.VMEM((C,), jnp.float32),
            pltpu.VMEM((C,), jnp.float32),
        ],
    )
    def k(T, ii, jj, D, bi, bj, bif, bjf, gi, gj, dc):
        w = lax.axis_index("s") * 2 + lax.axis_index("c")
        nch = jnp.where(w < extra, full + 1, full)

        def chunk_body(ci, carry):
            base = (ci * _NW + w) * C
            pltpu.sync_copy(ii.at[pl.ds(base, C)], bi)
            pltpu.sync_copy(jj.at[pl.ds(base, C)], bj)
            for f in range(4):
                def shift(g, c2):
                    sl = pl.ds(g * 16, 16)
                    bif[sl] = bi[sl] + f
                    bjf[sl] = bj[sl] + f
                    return c2

                lax.fori_loop(0, C // 16, shift, 0, unroll=4)
                pltpu.sync_copy(T.at[bif], gi)
                pltpu.sync_copy(T.at[bjf], gj)

                def sub(g, c2):
                    sl = pl.ds(g * 16, 16)
                    dc[sl] = gj[sl] - gi[sl]
                    return c2

                lax.fori_loop(0, C // 16, sub, 0, unroll=4)
                pltpu.sync_copy(dc, D.at[pl.ds(f * E + base, C)])
            return carry

        lax.fori_loop(0, nch, chunk_body, 0)

    return k(T1, i4, j4)


def _mlp_body(d_ref, w1_ref, b1_ref, w2_ref, b2_ref, w3_ref, b3_ref,
              ox_ref, oy_ref, nx_ref, ny_ref):
    D = d_ref[...]
    dx = D[0:1, :]
    dy = D[1:2, :]
    vx = D[2:3, :]
    vy = D[3:4, :]
    d2 = dx * dx + dy * dy
    dist = jnp.maximum(jnp.sqrt(d2), 1e-8)
    inv = 1.0 / dist
    nx = dx * inv
    ny = dy * inv
    app = (dx * vx + dy * vy) * inv
    X = jnp.concatenate([dist, app], axis=0)
    z1 = jax.lax.dot_general(w1_ref[...], X, (((0,), (0,)), ((), ())),
                             preferred_element_type=jnp.float32)
    z1 = z1 + b1_ref[...]
    h1 = z1 * (1.0 / (1.0 + jnp.exp(-z1)))
    z2 = jax.lax.dot_general(w2_ref[...], h1, (((0,), (0,)), ((), ())),
                             preferred_element_type=jnp.float32)
    z2 = z2 + b2_ref[...]
    h2 = z2 * (1.0 / (1.0 + jnp.exp(-z2)))
    f = jax.lax.dot_general(w3_ref[...], h2, (((0,), (0,)), ((), ())),
                            preferred_element_type=jnp.float32)
    f = f + b3_ref[...]
    ix = f * nx
    iy = f * ny
    ox_ref[...] = ix
    oy_ref[...] = iy
    nx_ref[...] = -ix
    ny_ref[...] = -iy


def _edge_mlp(Dm, W1, b1, W2, b2, W3, b3):
    E = Dm.shape[1]
    be = _BE if E % _BE == 0 else E
    grid = E // be

    def _w(shape):
        return pl.BlockSpec(shape, lambda e: (0, 0))

    ospec = pl.BlockSpec((1, be), lambda e: (0, e))
    oshape = jax.ShapeDtypeStruct((1, E), jnp.float32)
    return pl.pallas_call(
        _mlp_body,
        grid=(grid,),
        in_specs=[
            pl.BlockSpec((4, be), lambda e: (0, e)),
            _w((2, 16)), _w((16, 1)), _w((16, 16)), _w((16, 1)),
            _w((16, 1)), _w((1, 1)),
        ],
        out_specs=[ospec, ospec, ospec, ospec],
        out_shape=[oshape, oshape, oshape, oshape],
    )(Dm, W1, b1.reshape(16, 1), W2, b2.reshape(16, 1), W3,
      b3.reshape(1, 1))


def _sc_scatter(N, impx, impy, nimpx, nimpy, ih, jh, zeros_n):
    E = ih.shape[0]
    C = _C1
    nchunks = E // C
    full, extra = divmod(nchunks, _NW)
    half = _NW // 2
    zc = 6256
    zl = N - zc * (half - 1)
    mesh = plsc.VectorSubcoreMesh(core_axis_name="c", subcore_axis_name="s")

    @functools.partial(
        pl.kernel,
        mesh=mesh,
        compiler_params=pltpu.CompilerParams(use_tc_tiling_on_sc=False),
        out_type=[
            jax.ShapeDtypeStruct((2, N), jnp.float32),
            jax.ShapeDtypeStruct((2, N), jnp.float32),
        ],
        scratch_types=[
            pltpu.VMEM((C,), jnp.int32),
            pltpu.VMEM((C,), jnp.int32),
            pltpu.VMEM((C,), jnp.float32),
            pltpu.VMEM((C,), jnp.float32),
            pltpu.VMEM((C,), jnp.float32),
            pltpu.VMEM((C,), jnp.float32),
            pltpu.VMEM_SHARED((N,), jnp.float32),
            pltpu.VMEM_SHARED((N,), jnp.float32),
        ],
    )
    def k(ix, iy, nx, ny, ii, jj, zz, PX, PY,
          bi, bj, vx, vy, wx, wy, accx, accy):
        c = lax.axis_index("c")
        sc = lax.axis_index("s")
        w = sc * 2 + c
        nch = jnp.where(w < extra, full + 1, full)
        off = sc * zc

        @pl.when(sc < half - 1)
        def _zero_full():
            pltpu.sync_copy(zz.at[pl.ds(off, zc)], accx.at[pl.ds(off, zc)])
            pltpu.sync_copy(zz.at[pl.ds(off, zc)], accy.at[pl.ds(off, zc)])

        @pl.when(sc == half - 1)
        def _zero_last():
            pltpu.sync_copy(zz.at[pl.ds(off, zl)], accx.at[pl.ds(off, zl)])
            pltpu.sync_copy(zz.at[pl.ds(off, zl)], accy.at[pl.ds(off, zl)])

        plsc.subcore_barrier()

        def chunk_body(ci, carry):
            base = (ci * _NW + w) * C
            pltpu.sync_copy(ii.at[pl.ds(base, C)], bi)
            pltpu.sync_copy(jj.at[pl.ds(base, C)], bj)
            pltpu.sync_copy(ix.at[pl.ds(base, C)], vx)
            pltpu.sync_copy(iy.at[pl.ds(base, C)], vy)
            pltpu.sync_copy(nx.at[pl.ds(base, C)], wx)
            pltpu.sync_copy(ny.at[pl.ds(base, C)], wy)
            pltpu.sync_copy(vx, accx.at[bi], add=True)
            pltpu.sync_copy(vy, accy.at[bi], add=True)
            pltpu.sync_copy(wx, accx.at[bj], add=True)
            pltpu.sync_copy(wy, accy.at[bj], add=True)
            return carry

        lax.fori_loop(0, nch, chunk_body, 0)
        plsc.subcore_barrier()

        @pl.when(sc < half - 1)
        def _out_full():
            pltpu.sync_copy(accx.at[pl.ds(off, zc)], PX.at[c, pl.ds(off, zc)])
            pltpu.sync_copy(accy.at[pl.ds(off, zc)], PY.at[c, pl.ds(off, zc)])

        @pl.when(sc == half - 1)
        def _out_last():
            pltpu.sync_copy(accx.at[pl.ds(off, zl)], PX.at[c, pl.ds(off, zl)])
            pltpu.sync_copy(accy.at[pl.ds(off, zl)], PY.at[c, pl.ds(off, zl)])

    return k(impx, impy, nimpx, nimpy, ih, jh, zeros_n)


def _combine_body(s_ref, px_ref, py_ref, o_ref):
    ax = px_ref[0:1, :] + px_ref[1:2, :]
    ay = py_ref[0:1, :] + py_ref[1:2, :]
    BN = ax.shape[1]
    tx = jnp.transpose(ax)
    ty = jnp.transpose(ay)
    z = jnp.zeros((BN, 2), jnp.float32)
    o_ref[...] = s_ref[...] + jnp.concatenate([z, tx, ty], axis=1)


def _combine(s2d, PX, PY):
    N = s2d.shape[0]
    bn = 12800
    grid = N // bn
    return pl.pallas_call(
        _combine_body,
        grid=(grid,),
        in_specs=[
            pl.BlockSpec((bn, 4), lambda n: (n, 0)),
            pl.BlockSpec((2, bn), lambda n: (0, n)),
            pl.BlockSpec((2, bn), lambda n: (0, n)),
        ],
        out_specs=pl.BlockSpec((bn, 4), lambda n: (n, 0)),
        out_shape=jax.ShapeDtypeStruct((N, 4), jnp.float32),
    )(s2d, PX, PY)


def kernel(s, i, j, W1, b1, W2, b2, W3, b3):
    s2d = s[0]
    N = s2d.shape[0]
    E = i.shape[0]
    T1 = s2d.reshape(4 * N)
    Dm = _sc_gather(T1, i * 4, j * 4).reshape(4, E)
    impx, impy, nimpx, nimpy = _edge_mlp(Dm, W1, b1, W2, b2, W3, b3)
    zeros_n = jnp.zeros((N,), jnp.float32)
    PX, PY = _sc_scatter(N, impx.reshape(E), impy.reshape(E),
                         nimpx.reshape(E), nimpy.reshape(E), i, j, zeros_n)
    npad = (-N) % 12800
    out = _combine(jnp.pad(s2d, ((0, npad), (0, 0))),
                   jnp.pad(PX, ((0, 0), (0, npad))),
                   jnp.pad(PY, ((0, 0), (0, npad))))
    return out[:N][None]


# speedup vs baseline: 24.0321x; 1.1011x over previous
import functools

import jax
import jax.numpy as jnp
from jax import lax
from jax.experimental import pallas as pl
from jax.experimental.pallas import tpu as pltpu
from jax.experimental.pallas import tpu_sc as plsc

_BE = 12800
_C1 = 6400
_NW = 32


def _sc_gather(T1, i4, j4):
    E = i4.shape[0]
    C = _C1
    nchunks = E // C
    full, extra = divmod(nchunks, _NW)
    mesh = plsc.VectorSubcoreMesh(core_axis_name="c", subcore_axis_name="s")

    @functools.partial(
        pl.kernel,
        mesh=mesh,
        compiler_params=pltpu.CompilerParams(use_tc_tiling_on_sc=False),
        out_type=jax.ShapeDtypeStruct((4 * E,), jnp.float32),
        scratch_types=[
            pltpu.VMEM((C,), jnp.int32),
            pltpu.VMEM((C,), jnp.int32),
            pltpu.VMEM((C,), jnp.int32),
            pltpu.VMEM((C,), jnp.int32),
            pltpu.VMEM((C,), jnp.float32),
            pltpu.VMEM((C,), jnp.float32),
            pltpu.VMEM((4, C), jnp.float32),
            pltpu.SemaphoreType.DMA,
            pltpu.SemaphoreType.DMA,
            pltpu.SemaphoreType.DMA,
        ],
    )
    def k(T, ii, jj, D, bi, bj, bif, bjf, gi, gj, dc, s0, s1, s2):
        w = lax.axis_index("s") * 2 + lax.axis_index("c")
        nch = jnp.where(w < extra, full + 1, full)

        def chunk_body(ci, carry):
            base = (ci * _NW + w) * C
            a0 = pltpu.async_copy(ii.at[pl.ds(base, C)], bi, s0)
            a1 = pltpu.async_copy(jj.at[pl.ds(base, C)], bj, s1)
            a0.wait()
            a1.wait()
            wd = []
            for f in range(4):
                def shift(g, c2):
                    sl = pl.ds(g * 16, 16)
                    bif[sl] = bi[sl] + f
                    bjf[sl] = bj[sl] + f
                    return c2

                lax.fori_loop(0, C // 16, shift, 0, unroll=4)
                g0 = pltpu.async_copy(T.at[bif], gi, s0)
                g1 = pltpu.async_copy(T.at[bjf], gj, s1)
                g0.wait()
                g1.wait()

                def sub(g, c2):
                    sl = pl.ds(g * 16, 16)
                    dc[f, sl] = gj[sl] - gi[sl]
                    return c2

                lax.fori_loop(0, C // 16, sub, 0, unroll=4)
                wd.append(pltpu.async_copy(
                    dc.at[f], D.at[pl.ds(f * E + base, C)], s2))
            for a in wd:
                a.wait()
            return carry

        lax.fori_loop(0, nch, chunk_body, 0)

    return k(T1, i4, j4)


def _mlp_body(d_ref, w1_ref, b1_ref, w2_ref, b2_ref, w3_ref, b3_ref,
              ox_ref, oy_ref, nx_ref, ny_ref):
    D = d_ref[...]
    dx = D[0:1, :]
    dy = D[1:2, :]
    vx = D[2:3, :]
    vy = D[3:4, :]
    d2 = dx * dx + dy * dy
    dist = jnp.maximum(jnp.sqrt(d2), 1e-8)
    inv = 1.0 / dist
    nx = dx * inv
    ny = dy * inv
    app = (dx * vx + dy * vy) * inv
    X = jnp.concatenate([dist, app], axis=0)
    z1 = jax.lax.dot_general(w1_ref[...], X, (((0,), (0,)), ((), ())),
                             preferred_element_type=jnp.float32)
    z1 = z1 + b1_ref[...]
    h1 = z1 * (1.0 / (1.0 + jnp.exp(-z1)))
    z2 = jax.lax.dot_general(w2_ref[...], h1, (((0,), (0,)), ((), ())),
                             preferred_element_type=jnp.float32)
    z2 = z2 + b2_ref[...]
    h2 = z2 * (1.0 / (1.0 + jnp.exp(-z2)))
    f = jax.lax.dot_general(w3_ref[...], h2, (((0,), (0,)), ((), ())),
                            preferred_element_type=jnp.float32)
    f = f + b3_ref[...]
    ix = f * nx
    iy = f * ny
    ox_ref[...] = ix
    oy_ref[...] = iy
    nx_ref[...] = -ix
    ny_ref[...] = -iy


def _edge_mlp(Dm, W1, b1, W2, b2, W3, b3):
    E = Dm.shape[1]
    be = _BE if E % _BE == 0 else E
    grid = E // be

    def _w(shape):
        return pl.BlockSpec(shape, lambda e: (0, 0))

    ospec = pl.BlockSpec((1, be), lambda e: (0, e))
    oshape = jax.ShapeDtypeStruct((1, E), jnp.float32)
    return pl.pallas_call(
        _mlp_body,
        grid=(grid,),
        in_specs=[
            pl.BlockSpec((4, be), lambda e: (0, e)),
            _w((2, 16)), _w((16, 1)), _w((16, 16)), _w((16, 1)),
            _w((16, 1)), _w((1, 1)),
        ],
        out_specs=[ospec, ospec, ospec, ospec],
        out_shape=[oshape, oshape, oshape, oshape],
    )(Dm, W1, b1.reshape(16, 1), W2, b2.reshape(16, 1), W3,
      b3.reshape(1, 1))


def _sc_scatter(N, impx, impy, nimpx, nimpy, ih, jh, zeros_n):
    E = ih.shape[0]
    C = _C1
    nchunks = E // C
    full, extra = divmod(nchunks, _NW)
    half = _NW // 2
    zc = 6256
    zl = N - zc * (half - 1)
    mesh = plsc.VectorSubcoreMesh(core_axis_name="c", subcore_axis_name="s")

    @functools.partial(
        pl.kernel,
        mesh=mesh,
        compiler_params=pltpu.CompilerParams(use_tc_tiling_on_sc=False),
        out_type=[
            jax.ShapeDtypeStruct((2, N), jnp.float32),
            jax.ShapeDtypeStruct((2, N), jnp.float32),
        ],
        scratch_types=[
            pltpu.VMEM((C,), jnp.int32),
            pltpu.VMEM((C,), jnp.int32),
            pltpu.VMEM((C,), jnp.float32),
            pltpu.VMEM((C,), jnp.float32),
            pltpu.VMEM((C,), jnp.float32),
            pltpu.VMEM((C,), jnp.float32),
            pltpu.VMEM_SHARED((N,), jnp.float32),
            pltpu.VMEM_SHARED((N,), jnp.float32),
            pltpu.SemaphoreType.DMA,
            pltpu.SemaphoreType.DMA,
            pltpu.SemaphoreType.DMA,
        ],
    )
    def k(ix, iy, nx, ny, ii, jj, zz, PX, PY,
          bi, bj, vx, vy, wx, wy, accx, accy, s0, s1, s2):
        c = lax.axis_index("c")
        sc = lax.axis_index("s")
        w = sc * 2 + c
        nch = jnp.where(w < extra, full + 1, full)
        off = sc * zc

        @pl.when(sc < half - 1)
        def _zero_full():
            pltpu.sync_copy(zz.at[pl.ds(off, zc)], accx.at[pl.ds(off, zc)])
            pltpu.sync_copy(zz.at[pl.ds(off, zc)], accy.at[pl.ds(off, zc)])

        @pl.when(sc == half - 1)
        def _zero_last():
            pltpu.sync_copy(zz.at[pl.ds(off, zl)], accx.at[pl.ds(off, zl)])
            pltpu.sync_copy(zz.at[pl.ds(off, zl)], accy.at[pl.ds(off, zl)])

        plsc.subcore_barrier()

        def chunk_body(ci, carry):
            base = (ci * _NW + w) * C
            sl = pl.ds(base, C)
            aa = [pltpu.async_copy(ii.at[sl], bi, s0),
                  pltpu.async_copy(jj.at[sl], bj, s1),
                  pltpu.async_copy(ix.at[sl], vx, s2),
                  pltpu.async_copy(iy.at[sl], vy, s0),
                  pltpu.async_copy(nx.at[sl], wx, s1),
                  pltpu.async_copy(ny.at[sl], wy, s2)]
            for a in aa:
                a.wait()
            pltpu.sync_copy(vx, accx.at[bi], add=True)
            pltpu.sync_copy(vy, accy.at[bi], add=True)
            pltpu.sync_copy(wx, accx.at[bj], add=True)
            pltpu.sync_copy(wy, accy.at[bj], add=True)
            return carry

        lax.fori_loop(0, nch, chunk_body, 0)
        plsc.subcore_barrier()

        @pl.when(sc < half - 1)
        def _out_full():
            pltpu.sync_copy(accx.at[pl.ds(off, zc)], PX.at[c, pl.ds(off, zc)])
            pltpu.sync_copy(accy.at[pl.ds(off, zc)], PY.at[c, pl.ds(off, zc)])

        @pl.when(sc == half - 1)
        def _out_last():
            pltpu.sync_copy(accx.at[pl.ds(off, zl)], PX.at[c, pl.ds(off, zl)])
            pltpu.sync_copy(accy.at[pl.ds(off, zl)], PY.at[c, pl.ds(off, zl)])

    return k(impx, impy, nimpx, nimpy, ih, jh, zeros_n)


def _combine_body(s_ref, px_ref, py_ref, o_ref):
    ax = px_ref[0:1, :] + px_ref[1:2, :]
    ay = py_ref[0:1, :] + py_ref[1:2, :]
    BN = ax.shape[1]
    tx = jnp.transpose(ax)
    ty = jnp.transpose(ay)
    z = jnp.zeros((BN, 2), jnp.float32)
    o_ref[...] = s_ref[...] + jnp.concatenate([z, tx, ty], axis=1)


def _combine(s2d, PX, PY):
    N = s2d.shape[0]
    bn = 12800
    grid = N // bn
    return pl.pallas_call(
        _combine_body,
        grid=(grid,),
        in_specs=[
            pl.BlockSpec((bn, 4), lambda n: (n, 0)),
            pl.BlockSpec((2, bn), lambda n: (0, n)),
            pl.BlockSpec((2, bn), lambda n: (0, n)),
        ],
        out_specs=pl.BlockSpec((bn, 4), lambda n: (n, 0)),
        out_shape=jax.ShapeDtypeStruct((N, 4), jnp.float32),
    )(s2d, PX, PY)


def kernel(s, i, j, W1, b1, W2, b2, W3, b3):
    s2d = s[0]
    N = s2d.shape[0]
    E = i.shape[0]
    T1 = s2d.reshape(4 * N)
    Dm = _sc_gather(T1, i * 4, j * 4).reshape(4, E)
    impx, impy, nimpx, nimpy = _edge_mlp(Dm, W1, b1, W2, b2, W3, b3)
    zeros_n = jnp.zeros((N,), jnp.float32)
    PX, PY = _sc_scatter(N, impx.reshape(E), impy.reshape(E),
                         nimpx.reshape(E), nimpy.reshape(E), i, j, zeros_n)
    npad = (-N) % 12800
    out = _combine(jnp.pad(s2d, ((0, npad), (0, 0))),
                   jnp.pad(PX, ((0, 0), (0, npad))),
                   jnp.pad(PY, ((0, 0), (0, npad))))
    return out[:N][None]
